# baseline (speedup 1.0000x reference)
import jax
import jax.numpy as jnp
from jax.experimental import pallas as pl
from jax.experimental.pallas import tpu as pltpu

_B, _N, _D = 8, 512, 320
_E = 16
_K = 8192
_QB = 2048
_BN = _B * _N
_GRID = _BN // _QB


def _rpq_kernel(x_ref, rp_ref, cbt_ref, out_ref):
    xq = x_ref[...].reshape(_QB, _D)
    proj = jnp.dot(xq, rp_ref[...], preferred_element_type=jnp.float32)
    xnorm = jnp.sqrt(jnp.sum(proj * proj, axis=1, keepdims=True))
    xn = proj / jnp.maximum(xnorm, 1e-12)

    cbt = cbt_ref[...]
    cnorm = jnp.sqrt(jnp.sum(cbt * cbt, axis=0, keepdims=True))
    cn = cbt / jnp.maximum(cnorm, 1e-12)
    c2 = jnp.sum(cn * cn, axis=0, keepdims=True)

    xa = jnp.concatenate([xn, jnp.ones((xn.shape[0], 1), jnp.float32)], axis=1)
    cnb = jnp.concatenate([cn, -0.5 * c2], axis=0)
    scores = jnp.dot(xa, cnb, preferred_element_type=jnp.float32)

    arg = jnp.argmax(scores, axis=1)
    out_ref[0, 0, :] = arg.astype(jnp.int32)


def kernel(x, random_projection, codebook):
    cbt = codebook.T

    out = pl.pallas_call(
        _rpq_kernel,
        grid=(_GRID,),
        in_specs=[
            pl.BlockSpec((_QB // _N, _N, _D), lambda i: (i, 0, 0)),
            pl.BlockSpec((_D, _E), lambda i: (0, 0)),
            pl.BlockSpec((_E, _K), lambda i: (0, 0)),
        ],
        out_specs=pl.BlockSpec((1, 1, _QB), lambda i: (i, 0, 0)),
        out_shape=jax.ShapeDtypeStruct((_GRID, 1, _QB), jnp.int32),
        compiler_params=pltpu.CompilerParams(
            dimension_semantics=("parallel",),
        ),
    )(x, random_projection, cbt)
    return out.reshape(_B, _N)


# speedup vs baseline: 1.0918x; 1.0918x over previous
import jax
import jax.numpy as jnp
from jax.experimental import pallas as pl
from jax.experimental.pallas import tpu as pltpu

_B, _N, _D = 8, 512, 320
_E = 16
_K = 8192
_QB = 1024
_BN = _B * _N
_GRID = _BN // _QB


def _rpq_kernel(x_ref, rp_ref, cbt_ref, out_ref):
    xq = x_ref[...].reshape(_QB, _D)
    proj = jnp.dot(xq, rp_ref[...], preferred_element_type=jnp.float32)
    xnorm = jnp.sqrt(jnp.sum(proj * proj, axis=1, keepdims=True))
    xn = proj / jnp.maximum(xnorm, 1e-12)

    cbt = cbt_ref[...]
    cnorm = jnp.sqrt(jnp.sum(cbt * cbt, axis=0, keepdims=True))
    cn = cbt / jnp.maximum(cnorm, 1e-12)
    c2 = jnp.sum(cn * cn, axis=0, keepdims=True)

    xa = jnp.concatenate([xn, jnp.ones((xn.shape[0], 1), jnp.float32)], axis=1)
    cnb = jnp.concatenate([cn, -0.5 * c2], axis=0)
    scores = jnp.dot(xa, cnb, preferred_element_type=jnp.float32)

    arg = jnp.argmax(scores, axis=1)
    out_ref[0, 0, :] = arg.astype(jnp.int32)


def kernel(x, random_projection, codebook):
    cbt = codebook.T

    out = pl.pallas_call(
        _rpq_kernel,
        grid=(_GRID,),
        in_specs=[
            pl.BlockSpec((_QB // _N, _N, _D), lambda i: (i, 0, 0)),
            pl.BlockSpec((_D, _E), lambda i: (0, 0)),
            pl.BlockSpec((_E, _K), lambda i: (0, 0)),
        ],
        out_specs=pl.BlockSpec((1, 1, _QB), lambda i: (i, 0, 0)),
        out_shape=jax.ShapeDtypeStruct((_GRID, 1, _QB), jnp.int32),
        compiler_params=pltpu.CompilerParams(
            dimension_semantics=("parallel",),
            allow_input_fusion=[False, False, True],
            vmem_limit_bytes=128 * 1024 * 1024,
        ),
    )(x, random_projection, cbt)
    return out.reshape(_B, _N)
